# Initial kernel scaffold; baseline (speedup 1.0000x reference)
import jax
import jax.numpy as jnp
from jax.experimental import pallas as pl


def kernel(pre, gt, acc_sum):
    raise NotImplementedError("write your pallas kernel here")


# speedup vs baseline: 2.4087x; 2.4087x over previous
import functools
import jax
import jax.numpy as jnp
from jax.experimental import pallas as pl
from jax.experimental.pallas import tpu as pltpu

_BINS = 10
_LAST_EDGE = float(jnp.float32(1.0) + jnp.float32(1e-6))


def _body(acc_sum_ref, pre_ref, gt_ref, out_ref, sacc_ref):
    i = pl.program_id(0)
    n_steps = pl.num_programs(0)

    @pl.when(i == 0)
    def _init():
        for k in range(_BINS):
            sacc_ref[k] = jnp.float32(0.0)

    p = pre_ref[...]
    t = gt_ref[...]
    g = jnp.abs(p - t)
    b = jnp.minimum((g * jnp.float32(10.0)).astype(jnp.int32), _BINS - 1)
    x = jnp.where(t == jnp.float32(1.0), p, jnp.float32(1.0) - p)
    term = jnp.log(x)
    term = jnp.where(g < jnp.float32(_LAST_EDGE), term, jnp.float32(0.0))
    for k in range(_BINS):
        sacc_ref[k] += jnp.sum(jnp.where(b == k, term, jnp.float32(0.0)))

    @pl.when(i == n_steps - 1)
    def _final():
        total = jnp.float32(0.0)
        n = jnp.float32(0.0)
        for k in range(_BINS):
            s = sacc_ref[k]
            has = (s < jnp.float32(0.0)).astype(jnp.float32)
            n = n + has
            total = total + has * s / acc_sum_ref[k]
        out_ref[0] = total / jnp.maximum(n, jnp.float32(1.0))


def kernel(pre, gt, acc_sum):
    rows = pre.size // 128
    pre2 = pre.reshape(rows, 128)
    gt2 = gt.reshape(rows, 128)
    block_rows = 2048
    grid = rows // block_rows
    out = pl.pallas_call(
        _body,
        grid=(grid,),
        in_specs=[
            pl.BlockSpec(memory_space=pltpu.SMEM),
            pl.BlockSpec((block_rows, 128), lambda i: (i, 0)),
            pl.BlockSpec((block_rows, 128), lambda i: (i, 0)),
        ],
        out_specs=pl.BlockSpec(memory_space=pltpu.SMEM),
        out_shape=jax.ShapeDtypeStruct((1,), jnp.float32),
        scratch_shapes=[pltpu.SMEM((_BINS,), jnp.float32)],
    )(acc_sum, pre2, gt2)
    return out[0]
